# Initial kernel scaffold; baseline (speedup 1.0000x reference)
import jax
import jax.numpy as jnp
from jax.experimental import pallas as pl


def kernel(loc_preds, cls_preds, priorbox, targets):
    raise NotImplementedError("write your pallas kernel here")


# speedup vs baseline: 81.7394x; 81.7394x over previous
import jax
import jax.numpy as jnp
from jax.experimental import pallas as pl
from jax.experimental.pallas import tpu as pltpu

_B = 32
_P = 8732
_C = 21
_G = 10
_CH = 384
_NSTEP = 23
_PPAD = _CH * _NSTEP
_THRESH = 0.5
_NEGPOS = 3
_V0 = 0.1
_V1 = 0.2


def _match_body(pb_ref, tg_ref, loct_ref, conf_ref):
    f32 = jnp.float32
    cx = pb_ref[0:1, :]
    cy = pb_ref[1:2, :]
    w = pb_ref[2:3, :]
    h = pb_ref[3:4, :]
    px1 = cx - w * 0.5
    py1 = cy - h * 0.5
    px2 = cx + w * 0.5
    py2 = cy + h * 0.5
    area_b = (px2 - px1) * (py2 - py1)

    tg = tg_ref[...]
    lane = jax.lax.broadcasted_iota(jnp.int32, (_B, _P), 1)

    bto = jnp.full((_B, _P), -1.0, f32)
    bti = jnp.zeros((_B, _P), jnp.int32)
    bpids = []
    for g in range(_G):
        tx1 = tg[0][:, g:g + 1]
        ty1 = tg[1][:, g:g + 1]
        tx2 = tg[2][:, g:g + 1]
        ty2 = tg[3][:, g:g + 1]
        iw = jnp.maximum(jnp.minimum(tx2, px2) - jnp.maximum(tx1, px1), 0.0)
        ih = jnp.maximum(jnp.minimum(ty2, py2) - jnp.maximum(ty1, py1), 0.0)
        inter = iw * ih
        area_a = (tx2 - tx1) * (ty2 - ty1)
        ov = inter / (area_a + area_b - inter + 1e-8)
        upd = ov > bto
        bti = jnp.where(upd, g, bti)
        bto = jnp.where(upd, ov, bto)
        mx = jnp.max(ov, axis=1, keepdims=True)
        bpid = jnp.min(jnp.where(ov >= mx, lane, _P), axis=1, keepdims=True)
        bpids.append(bpid)

    for g in range(_G):
        m = lane == bpids[g]
        bto = jnp.where(m, 2.0, bto)
        bti = jnp.where(m, g, bti)

    conf_f = jnp.zeros((_B, _P), f32)
    mx1 = jnp.zeros((_B, _P), f32)
    my1 = jnp.zeros((_B, _P), f32)
    mx2 = jnp.zeros((_B, _P), f32)
    my2 = jnp.zeros((_B, _P), f32)
    for g in range(_G):
        m = bti == g
        conf_f = jnp.where(m, tg[4][:, g:g + 1], conf_f)
        mx1 = jnp.where(m, tg[0][:, g:g + 1], mx1)
        my1 = jnp.where(m, tg[1][:, g:g + 1], my1)
        mx2 = jnp.where(m, tg[2][:, g:g + 1], mx2)
        my2 = jnp.where(m, tg[3][:, g:g + 1], my2)
    conf_f = jnp.where(bto < _THRESH, 0.0, conf_f)
    conf_ref[...] = conf_f.astype(jnp.int32)

    loct_ref[0] = ((mx1 + mx2) * 0.5 - cx) / (_V0 * w)
    loct_ref[1] = ((my1 + my2) * 0.5 - cy) / (_V0 * h)
    loct_ref[2] = jnp.log(jnp.maximum((mx2 - mx1) / w, 1e-8)) * (1.0 / _V1)
    loct_ref[3] = jnp.log(jnp.maximum((my2 - my1) / h, 1e-8)) * (1.0 / _V1)


def _stream_body(xt_ref, lt_ref, loct_ref, conf_ref, lossc_ref, acc_ref, np_ref):
    j = pl.program_id(0)
    lane = jax.lax.broadcasted_iota(jnp.int32, (_B, _CH), 1) + j * _CH
    valid = lane < _P
    conf = conf_ref[...]
    pos = (conf > 0) & valid

    sumexp = jnp.zeros((_B, _CH), jnp.float32)
    xg = jnp.zeros((_B, _CH), jnp.float32)
    for c in range(_C):
        xc = xt_ref[c]
        sumexp = sumexp + jnp.exp(xc)
        xg = jnp.where(conf == c, xc, xg)
    ce = jnp.log(sumexp) - xg
    ce = jnp.where(valid, ce, 0.0)
    lossc_ref[...] = jnp.where(pos, 0.0, ce)

    sl = jnp.zeros((_B, _CH), jnp.float32)
    for c in range(4):
        d = lt_ref[c] - loct_ref[c]
        a = jnp.abs(d)
        sl = sl + jnp.where(a < 1.0, 0.5 * d * d, a - 0.5)
    contrib = jnp.where(pos, ce + sl, 0.0)

    @pl.when(j == 0)
    def _init():
        acc_ref[...] = jnp.zeros((_B, _CH), jnp.float32)
        np_ref[...] = jnp.zeros((_B, _CH), jnp.float32)

    acc_ref[...] += contrib
    np_ref[...] += jnp.where(pos, 1.0, 0.0)


def _select_body(lossc_ref, acc_ref, np_ref, out_ref):
    np_b = jnp.sum(np_ref[...], axis=1, keepdims=True)
    k = jnp.minimum(_NEGPOS * np_b, float(_P - 1))
    n_tot = jnp.sum(np_b)
    base = jnp.sum(acc_ref[...])

    v = lossc_ref[...]
    vb = jax.lax.bitcast_convert_type(v, jnp.int32)

    def body(_, carry):
        lo, hi = carry
        mid = lo + jax.lax.shift_right_logical(hi - lo, 1)
        cnt = jnp.sum(jnp.where(vb >= mid, 1.0, 0.0), axis=1, keepdims=True)
        ge = cnt >= k
        return jnp.where(ge, mid, lo), jnp.where(ge, hi, mid)

    lo0 = jnp.zeros((_B, 1), jnp.int32)
    hi0 = jnp.full((_B, 1), jnp.int32(0x7F800001))
    t_bits, _ = jax.lax.fori_loop(0, 31, body, (lo0, hi0))
    t_val = jax.lax.bitcast_convert_type(t_bits, jnp.float32)

    gt = vb > t_bits
    m = jnp.sum(jnp.where(gt, 1.0, 0.0), axis=1, keepdims=True)
    s = jnp.sum(jnp.where(gt, v, 0.0), axis=1, keepdims=True)
    neg = s + (k - m) * t_val
    neg = jnp.where(k >= 1.0, neg, 0.0)

    denom = jnp.maximum(n_tot, 1.0)
    out_ref[...] = ((base + jnp.sum(neg)) / denom).reshape(1, 1)


def _build(interpret=False):
    f32 = jnp.float32
    match_call = pl.pallas_call(
        _match_body,
        out_shape=[
            jax.ShapeDtypeStruct((4, _B, _P), f32),
            jax.ShapeDtypeStruct((_B, _P), jnp.int32),
        ],
        interpret=interpret,
    )
    stream_call = pl.pallas_call(
        _stream_body,
        grid=(_NSTEP,),
        in_specs=[
            pl.BlockSpec((_C, _B, _CH), lambda j: (0, 0, j)),
            pl.BlockSpec((4, _B, _CH), lambda j: (0, 0, j)),
            pl.BlockSpec((4, _B, _CH), lambda j: (0, 0, j)),
            pl.BlockSpec((_B, _CH), lambda j: (0, j)),
        ],
        out_specs=[
            pl.BlockSpec((_B, _CH), lambda j: (0, j)),
            pl.BlockSpec((_B, _CH), lambda j: (0, 0)),
            pl.BlockSpec((_B, _CH), lambda j: (0, 0)),
        ],
        out_shape=[
            jax.ShapeDtypeStruct((_B, _PPAD), f32),
            jax.ShapeDtypeStruct((_B, _CH), f32),
            jax.ShapeDtypeStruct((_B, _CH), f32),
        ],
        interpret=interpret,
    )
    select_call = pl.pallas_call(
        _select_body,
        out_shape=jax.ShapeDtypeStruct((1, 1), f32),
        interpret=interpret,
    )
    return match_call, stream_call, select_call


def _loss(loc_preds, cls_preds, priorbox, targets, interpret=False):
    match_call, stream_call, select_call = _build(interpret)
    xt = jnp.transpose(cls_preds, (2, 0, 1))
    lt = jnp.transpose(loc_preds, (2, 0, 1))
    pbt = jnp.transpose(priorbox, (1, 0))
    tgt = jnp.transpose(targets, (2, 0, 1))
    loct, conf = match_call(pbt, tgt)
    lossc, acc, npf = stream_call(xt, lt, loct, conf)
    out = select_call(lossc, acc, npf)
    return out.reshape(())


def kernel(loc_preds, cls_preds, priorbox, targets):
    return _loss(loc_preds, cls_preds, priorbox, targets)
